# baseline (speedup 1.0000x reference)
import functools

import jax
import jax.numpy as jnp
from jax import lax
from jax.experimental import pallas as pl
from jax.experimental.pallas import tpu as pltpu
from jax.experimental.pallas import tpu_sc as plsc

B = 16384
N_NUM = 13
N_CAT = 26
HC = N_CAT // 2
VOCAB = 100000
EMB = 16
LANES = 16

NC = 2
NS = 16
NW = NC * NS
RPW = B // NW
IDXW = 128
IDXR_H = RPW * HC // IDXW
SPH = RPW * HC
VPAD = 100352

_MESH = plsc.VectorSubcoreMesh(core_axis_name="c", subcore_axis_name="s")
_PARAMS = pltpu.CompilerParams(
    needs_layout_passes=False, use_tc_tiling_on_sc=False)


def _scores_half_body(base, tabt_ref, w_ref, out_ref):
    c = pl.program_id(0)
    blk = tabt_ref[0]
    w = w_ref[base + c]
    s = jnp.sum(blk * w[:, None], axis=0)
    out_ref[...] = jnp.concatenate(
        [s, jnp.zeros((VPAD - VOCAB,), jnp.float32)])


def _scores_half(tabt, wcat, base):
    return pl.pallas_call(
        functools.partial(_scores_half_body, base),
        grid=(HC,),
        in_specs=[
            pl.BlockSpec((1, EMB, VOCAB), lambda c: (c + base, 0, 0)),
            pl.BlockSpec((N_CAT, EMB), lambda c: (0, 0)),
        ],
        out_specs=pl.BlockSpec((VPAD,), lambda c: (c,)),
        out_shape=jax.ShapeDtypeStruct((HC * VPAD,), jnp.float32),
    )(tabt, wcat)


def _stage_gathers(scores_hbm, idx_hbm, wid, idx_v, sv_v, sem):
    pltpu.sync_copy(idx_hbm.at[pl.ds(wid * IDXR_H, IDXR_H), :], idx_v)

    def gather4(g, carry):
        copies = [
            pltpu.async_copy(
                scores_hbm.at[idx_v.at[g * 4 + j]],
                sv_v.at[pl.ds((g * 4 + j) * IDXW, IDXW)],
                sem,
            )
            for j in range(4)
        ]
        for cp in copies:
            cp.wait()
        return carry

    lax.fori_loop(0, IDXR_H // 4, gather4, 0)


def _gather1_body(scores_hbm, idx_hbm, num_hbm, wnum_hbm, out_hbm,
                  idx_v, sv_v, num_v, wnum_v, out_v, sem):
    wid = lax.axis_index("s") * NC + lax.axis_index("c")
    pltpu.sync_copy(wnum_hbm, wnum_v)
    pltpu.sync_copy(num_hbm.at[pl.ds(wid * RPW * LANES, RPW * LANES)], num_v)
    _stage_gathers(scores_hbm, idx_hbm, wid, idx_v, sv_v, sem)
    w_num = wnum_v[pl.ds(0, LANES)]
    iota = lax.iota(jnp.int32, LANES)
    mask = iota < HC
    zeros = jnp.zeros((LANES,), jnp.float32)

    def group_body(t, c2):
        r0 = t * LANES
        out_vec = zeros
        for jj in range(LANES):
            r = r0 + jj
            s1 = sv_v[pl.ds(r * HC, LANES)]
            acc = (jnp.where(mask, s1, zeros)
                   + num_v[pl.ds(r * LANES, LANES)] * w_num)
            tot = jnp.broadcast_to(jnp.sum(acc), (LANES,))
            out_vec = jnp.where(iota == jj, tot, out_vec)
        out_v[pl.ds(r0, LANES)] = out_vec
        return c2

    lax.fori_loop(0, RPW // LANES, group_body, 0)
    pltpu.sync_copy(out_v, out_hbm.at[pl.ds(wid * RPW, RPW)])


def _gather2_body(scores_hbm, idx_hbm, part_hbm, out_hbm,
                  idx_v, sv_v, part_v, out_v, sem):
    wid = lax.axis_index("s") * NC + lax.axis_index("c")
    pltpu.sync_copy(part_hbm.at[pl.ds(wid * RPW, RPW)], part_v)
    _stage_gathers(scores_hbm, idx_hbm, wid, idx_v, sv_v, sem)
    iota = lax.iota(jnp.int32, LANES)
    mask = iota < HC
    zeros = jnp.zeros((LANES,), jnp.float32)

    def group_body(t, c2):
        r0 = t * LANES
        out_vec = zeros
        for jj in range(LANES):
            r = r0 + jj
            s1 = sv_v[pl.ds(r * HC, LANES)]
            tot = jnp.broadcast_to(jnp.sum(jnp.where(mask, s1, zeros)),
                                   (LANES,))
            out_vec = jnp.where(iota == jj, tot, out_vec)
        out_v[pl.ds(r0, LANES)] = out_vec + part_v[pl.ds(r0, LANES)]
        return c2

    lax.fori_loop(0, RPW // LANES, group_body, 0)
    pltpu.sync_copy(out_v, out_hbm.at[pl.ds(wid * RPW, RPW)])


_gather1_call = functools.partial(
    pl.kernel,
    mesh=_MESH,
    out_type=jax.ShapeDtypeStruct((B,), jnp.float32),
    scratch_types=[
        pltpu.VMEM((IDXR_H, IDXW), jnp.int32),
        pltpu.VMEM((SPH + 2 * LANES,), jnp.float32),
        pltpu.VMEM((RPW * LANES,), jnp.float32),
        pltpu.VMEM((LANES,), jnp.float32),
        pltpu.VMEM((RPW,), jnp.float32),
        pltpu.SemaphoreType.DMA,
    ],
    compiler_params=_PARAMS,
)(_gather1_body)

_gather2_call = functools.partial(
    pl.kernel,
    mesh=_MESH,
    out_type=jax.ShapeDtypeStruct((B,), jnp.float32),
    scratch_types=[
        pltpu.VMEM((IDXR_H, IDXW), jnp.int32),
        pltpu.VMEM((SPH + 2 * LANES,), jnp.float32),
        pltpu.VMEM((RPW,), jnp.float32),
        pltpu.VMEM((RPW,), jnp.float32),
        pltpu.SemaphoreType.DMA,
    ],
    compiler_params=_PARAMS,
)(_gather2_body)


def kernel(clinical_numerical_input, clinical_categorical_input, tables, W1, b1, W2, b2):
    weff = (W1 @ W2)[:, 0]
    beff = (b1 @ W2 + b2)[0]
    wcat = weff[N_NUM:].reshape(N_CAT, EMB)
    wnum = jnp.concatenate([
        weff[:N_NUM], beff[None], jnp.zeros((2,), jnp.float32)])
    tabt = jnp.transpose(tables, (0, 2, 1))
    num_pad = jnp.concatenate([
        clinical_numerical_input,
        jnp.ones((B, 1), jnp.float32),
        jnp.zeros((B, 2), jnp.float32),
    ], axis=1).reshape(B * LANES)
    idx = (clinical_categorical_input
           + (jnp.arange(N_CAT, dtype=jnp.int32) % HC * VPAD)[None, :])
    idx1 = idx[:, :HC].reshape(B * HC // IDXW, IDXW)
    idx2 = idx[:, HC:].reshape(B * HC // IDXW, IDXW)
    scores1 = _scores_half(tabt, wcat, 0)
    part = _gather1_call(scores1, idx1, num_pad, wnum)
    scores2 = _scores_half(tabt, wcat, HC)
    out = _gather2_call(scores2, idx2, part)
    return out.reshape(B, 1)


# speedup vs baseline: 1.2705x; 1.2705x over previous
import functools

import jax
import jax.numpy as jnp
from jax import lax
from jax.experimental import pallas as pl
from jax.experimental.pallas import tpu as pltpu
from jax.experimental.pallas import tpu_sc as plsc

B = 16384
N_NUM = 13
N_CAT = 26
HC = N_CAT // 2
VOCAB = 100000
EMB = 16
LANES = 16

NC = 2
NS = 16
NW = NC * NS
RPW = B // NW
IDXW = 128
GPH = HC * RPW // IDXW
VPAD = 100352

_MESH = plsc.VectorSubcoreMesh(core_axis_name="c", subcore_axis_name="s")
_PARAMS = pltpu.CompilerParams(
    needs_layout_passes=False, use_tc_tiling_on_sc=False)


def _scores_half_body(base, tabt_ref, w_ref, out_ref):
    c = pl.program_id(0)
    blk = tabt_ref[0]
    w = w_ref[base + c]
    s = jnp.sum(blk * w[:, None], axis=0)
    out_ref[...] = jnp.concatenate(
        [s, jnp.zeros((VPAD - VOCAB,), jnp.float32)])


def _scores_half(tabt, wcat, base):
    return pl.pallas_call(
        functools.partial(_scores_half_body, base),
        grid=(HC,),
        in_specs=[
            pl.BlockSpec((1, EMB, VOCAB), lambda c: (c + base, 0, 0)),
            pl.BlockSpec((N_CAT, EMB), lambda c: (0, 0)),
        ],
        out_specs=pl.BlockSpec((VPAD,), lambda c: (c,)),
        out_shape=jax.ShapeDtypeStruct((HC * VPAD,), jnp.float32),
    )(tabt, wcat)


def _idx_body(cat_ref, out_ref):
    blk = cat_ref[...]
    c_iota = lax.broadcasted_iota(jnp.int32, (N_CAT, B), 0)
    local_c = jnp.where(c_iota < HC, c_iota, c_iota - HC)
    out_ref[...] = (blk + local_c * VPAD).reshape(N_CAT * B)


def _idx_flat(cat_t):
    return pl.pallas_call(
        _idx_body,
        in_specs=[pl.BlockSpec((N_CAT, B), lambda: (0, 0))],
        out_specs=pl.BlockSpec((N_CAT * B,), lambda: (0,)),
        out_shape=jax.ShapeDtypeStruct((N_CAT * B,), jnp.int32),
    )(cat_t)


def _numpart_body(numt_ref, w_ref, out_ref):
    acc = jnp.broadcast_to(w_ref[N_NUM], (B,))
    for n in range(N_NUM):
        acc = acc + numt_ref[n] * w_ref[n]
    out_ref[...] = acc


def _numpart(num_t, wnum):
    return pl.pallas_call(
        _numpart_body,
        in_specs=[
            pl.BlockSpec((N_NUM, B), lambda: (0, 0)),
            pl.BlockSpec((LANES,), lambda: (0,)),
        ],
        out_specs=pl.BlockSpec((B,), lambda: (0,)),
        out_shape=jax.ShapeDtypeStruct((B,), jnp.float32),
    )(num_t, wnum)


def _gather_half_body(cbase, scores_hbm, idx_hbm, base_hbm, out_hbm,
                      idx_v, sv_v, base_v, out_v, sem):
    wid = lax.axis_index("s") * NC + lax.axis_index("c")
    rpc = RPW // IDXW
    stages = [
        pltpu.async_copy(
            idx_hbm.at[pl.ds((cbase + c) * (B // IDXW) + wid * rpc, rpc), :],
            idx_v.at[pl.ds(c * rpc, rpc), :],
            sem,
        )
        for c in range(HC)
    ]
    pltpu.sync_copy(base_hbm.at[pl.ds(wid * RPW, RPW)], base_v)
    for cp in stages:
        cp.wait()

    def gather4(g, carry):
        copies = [
            pltpu.async_copy(
                scores_hbm.at[idx_v.at[g * 4 + j]],
                sv_v.at[pl.ds((g * 4 + j) * IDXW, IDXW)],
                sem,
            )
            for j in range(4)
        ]
        for cp in copies:
            cp.wait()
        return carry

    lax.fori_loop(0, GPH // 4, gather4, 0)

    def chunk(t, c2):
        acc = base_v[pl.ds(t * LANES, LANES)]
        for c in range(HC):
            acc = acc + sv_v[pl.ds(c * RPW + t * LANES, LANES)]
        out_v[pl.ds(t * LANES, LANES)] = acc
        return c2

    lax.fori_loop(0, RPW // LANES, chunk, 0)
    pltpu.sync_copy(out_v, out_hbm.at[pl.ds(wid * RPW, RPW)])


def _gather_half(cbase):
    return functools.partial(
        pl.kernel,
        mesh=_MESH,
        out_type=jax.ShapeDtypeStruct((B,), jnp.float32),
        scratch_types=[
            pltpu.VMEM((GPH, IDXW), jnp.int32),
            pltpu.VMEM((HC * RPW,), jnp.float32),
            pltpu.VMEM((RPW,), jnp.float32),
            pltpu.VMEM((RPW,), jnp.float32),
            pltpu.SemaphoreType.DMA,
        ],
        compiler_params=_PARAMS,
    )(functools.partial(_gather_half_body, cbase))


_gather1_call = _gather_half(0)
_gather2_call = _gather_half(HC)


def kernel(clinical_numerical_input, clinical_categorical_input, tables, W1, b1, W2, b2):
    weff = (W1 @ W2)[:, 0]
    beff = (b1 @ W2 + b2)[0]
    wcat = weff[N_NUM:].reshape(N_CAT, EMB)
    wnum = jnp.concatenate([
        weff[:N_NUM], beff[None], jnp.zeros((2,), jnp.float32)])
    tabt = jnp.transpose(tables, (0, 2, 1))
    cat_t = clinical_categorical_input.T
    num_t = clinical_numerical_input.T
    idxf = _idx_flat(cat_t).reshape(N_CAT * B // IDXW, IDXW)
    numpart = _numpart(num_t, wnum)
    scores1 = _scores_half(tabt, wcat, 0)
    part = _gather1_call(scores1, idxf, numpart)
    scores2 = _scores_half(tabt, wcat, HC)
    out = _gather2_call(scores2, idxf, part)
    return out.reshape(B, 1)
